# baseline (device time: 26636 ns/iter reference)
import jax
import jax.numpy as jnp
from jax import lax
from jax.experimental import pallas as pl
from jax.experimental.pallas import tpu as pltpu

N_DEV = 4
B = 2
SQ = 128
D = 512
H_LOC = 8
DH = 64
SCALE = 0.125
QROWS = B * SQ // N_DEV

_CompilerParams = getattr(pltpu, "CompilerParams", None) or getattr(
    pltpu, "TPUCompilerParams"
)

BF = jnp.bfloat16
F32 = jnp.float32


def _body(x_ref, wq_ref, wo_ref, k_hbm, v_hbm, out_ref,
          k_ref, v_ref, part_ref, rs_ref, agg_ref,
          kv_sems, rs_send_sems, rs_recv_sems, ag_send_sems, ag_recv_sems):
    my = lax.axis_index("i")

    kv_copies = []
    for b in range(B):
        for h in range(H_LOC):
            i = b * H_LOC + h
            ck = pltpu.make_async_copy(
                k_hbm.at[b, :, my * H_LOC + h, :], k_ref.at[i], kv_sems.at[0]
            )
            cv = pltpu.make_async_copy(
                v_hbm.at[b, :, my * H_LOC + h, :], v_ref.at[i], kv_sems.at[1]
            )
            ck.start()
            cv.start()
            kv_copies.append((ck, cv))

    barrier_sem = pltpu.get_barrier_semaphore()
    for d in range(1, N_DEV):
        peer = lax.rem(my + d, N_DEV)
        pl.semaphore_signal(
            barrier_sem, inc=1,
            device_id=(peer,), device_id_type=pl.DeviceIdType.MESH,
        )

    q2 = jnp.dot(
        lax.convert_element_type(x_ref[:], BF),
        lax.convert_element_type(wq_ref[:], BF),
        preferred_element_type=F32,
    )
    wo_bf = lax.convert_element_type(wo_ref[:], BF)

    rs_sends = {}
    for d in range(1, N_DEV):
        peer = lax.rem(my + d, N_DEV)
        rs_sends[d] = pltpu.make_async_remote_copy(
            src_ref=part_ref.at[pl.ds(peer * QROWS, QROWS), :],
            dst_ref=rs_ref.at[d],
            send_sem=rs_send_sems.at[d],
            recv_sem=rs_recv_sems.at[d],
            device_id=(peer,),
            device_id_type=pl.DeviceIdType.MESH,
        )

    for ck, cv in kv_copies:
        ck.wait()
        cv.wait()
    pl.semaphore_wait(barrier_sem, N_DEV - 1)

    for b in range(B):
        cols = []
        for h in range(H_LOC):
            i = b * H_LOC + h
            qbh = lax.convert_element_type(
                q2[b * SQ:(b + 1) * SQ, h * DH:(h + 1) * DH], BF
            )
            kbh = lax.convert_element_type(k_ref[i], BF)
            vbh = lax.convert_element_type(v_ref[i], BF)
            s = lax.dot_general(
                qbh, kbh, (((1,), (1,)), ((), ())),
                preferred_element_type=F32,
            ) * SCALE
            m = jnp.max(s, axis=1, keepdims=True)
            p = jnp.exp(s - m)
            l = jnp.sum(p, axis=1, keepdims=True)
            pb = lax.convert_element_type(p, BF)
            o = jnp.dot(pb, vbh, preferred_element_type=F32) / l
            cols.append(lax.convert_element_type(o, BF))
        attn_b = jnp.concatenate(cols, axis=1)
        part_ref[pl.ds(b * SQ, SQ), :] = lax.convert_element_type(
            jnp.dot(attn_b, wo_bf, preferred_element_type=F32), BF
        )
        for d in (2, 1, 3):
            peer = lax.rem(my + d, N_DEV)

            @pl.when(lax.div(peer, 2) == b)
            def _(rdma=rs_sends[d]):
                rdma.start()

    reduced = lax.convert_element_type(
        part_ref[pl.ds(my * QROWS, QROWS), :], F32
    )
    for d in (1, 3, 2):
        rs_sends[d].wait_recv()
        reduced = reduced + lax.convert_element_type(rs_ref[d], F32)

    rs_ref[0] = lax.convert_element_type(reduced, BF)
    ag_sends = {}
    for d in (2, 1, 3):
        peer = lax.rem(my + d, N_DEV)
        rdma = pltpu.make_async_remote_copy(
            src_ref=rs_ref.at[0],
            dst_ref=agg_ref.at[pl.ds(my * QROWS, QROWS), :],
            send_sem=ag_send_sems.at[d],
            recv_sem=ag_recv_sems.at[d],
            device_id=(peer,),
            device_id_type=pl.DeviceIdType.MESH,
        )
        rdma.start()
        ag_sends[d] = rdma
    agg_ref[pl.ds(my * QROWS, QROWS), :] = rs_ref[0]

    for d in (1, 3, 2):
        ag_sends[d].wait_recv()
    out_ref[:] = lax.convert_element_type(agg_ref[:], F32)
    out_ref[pl.ds(my * QROWS, QROWS), :] = reduced
    for d in range(1, N_DEV):
        rs_sends[d].wait_send()
        ag_sends[d].wait_send()


def kernel(x, Wq, Wo, K_ext, V_ext):
    out2 = pl.pallas_call(
        _body,
        out_shape=jax.ShapeDtypeStruct((B * SQ, D), F32),
        in_specs=[
            pl.BlockSpec(memory_space=pltpu.VMEM),
            pl.BlockSpec(memory_space=pltpu.VMEM),
            pl.BlockSpec(memory_space=pltpu.VMEM),
            pl.BlockSpec(memory_space=pl.ANY),
            pl.BlockSpec(memory_space=pl.ANY),
        ],
        out_specs=pl.BlockSpec(memory_space=pltpu.VMEM),
        scratch_shapes=[
            pltpu.VMEM((B * H_LOC, SQ, DH), F32),
            pltpu.VMEM((B * H_LOC, SQ, DH), F32),
            pltpu.VMEM((B * SQ, D), BF),
            pltpu.VMEM((N_DEV, QROWS, D), BF),
            pltpu.VMEM((B * SQ, D), BF),
            pltpu.SemaphoreType.DMA((2,)),
            pltpu.SemaphoreType.DMA((N_DEV,)),
            pltpu.SemaphoreType.DMA((N_DEV,)),
            pltpu.SemaphoreType.DMA((N_DEV,)),
            pltpu.SemaphoreType.DMA((N_DEV,)),
        ],
        compiler_params=_CompilerParams(collective_id=0),
    )(x.reshape(B * SQ, D), Wq, Wo, K_ext, V_ext)
    return out2.reshape(B, SQ, D)


# device time: 15708 ns/iter; 1.6957x vs baseline; 1.6957x over previous
import jax
import jax.numpy as jnp
from jax import lax
from jax.experimental import pallas as pl
from jax.experimental.pallas import tpu as pltpu

N_DEV = 4
B = 2
SQ = 128
D = 512
H_LOC = 8
DH = 64
SCALE = 0.125
QROWS = B * SQ // N_DEV

_CompilerParams = getattr(pltpu, "CompilerParams", None) or getattr(
    pltpu, "TPUCompilerParams"
)

BF = jnp.bfloat16
F32 = jnp.float32


def _body(x_ref, wq_ref, wo_ref, kt_ref, v_ref, out_ref,
          part_ref, rs_ref, agg_ref,
          rs_send_sems, rs_recv_sems, ag_send_sems, ag_recv_sems):
    my = lax.axis_index("i")

    barrier_sem = pltpu.get_barrier_semaphore()
    for d in range(1, N_DEV):
        peer = lax.rem(my + d, N_DEV)
        pl.semaphore_signal(
            barrier_sem, inc=1,
            device_id=(peer,), device_id_type=pl.DeviceIdType.MESH,
        )

    q2 = jnp.dot(
        lax.convert_element_type(x_ref[:], BF),
        lax.convert_element_type(wq_ref[:], BF),
        preferred_element_type=F32,
    )
    wo_bf = lax.convert_element_type(wo_ref[:], BF)

    rs_sends = {}
    for d in range(1, N_DEV):
        peer = lax.rem(my + d, N_DEV)
        rs_sends[d] = pltpu.make_async_remote_copy(
            src_ref=part_ref.at[pl.ds(peer * QROWS, QROWS), :],
            dst_ref=rs_ref.at[d],
            send_sem=rs_send_sems.at[d],
            recv_sem=rs_recv_sems.at[d],
            device_id=(peer,),
            device_id_type=pl.DeviceIdType.MESH,
        )

    pl.semaphore_wait(barrier_sem, N_DEV - 1)

    for b in range(B):
        cols = []
        for h in range(H_LOC):
            i = b * H_LOC + h
            qbh = lax.convert_element_type(
                q2[b * SQ:(b + 1) * SQ, h * DH:(h + 1) * DH], BF
            )
            kth = kt_ref[i]
            vbh = v_ref[i]
            s = jnp.dot(
                qbh, kth, preferred_element_type=F32
            ) * SCALE
            m = jnp.max(s, axis=1, keepdims=True)
            p = jnp.exp(s - m)
            l = jnp.sum(p, axis=1, keepdims=True)
            pb = lax.convert_element_type(p, BF)
            o = jnp.dot(pb, vbh, preferred_element_type=F32) / l
            cols.append(lax.convert_element_type(o, BF))
        attn_b = jnp.concatenate(cols, axis=1)
        part_ref[pl.ds(b * SQ, SQ), :] = lax.convert_element_type(
            jnp.dot(attn_b, wo_bf, preferred_element_type=F32), BF
        )
        for d in (2, 1, 3):
            peer = lax.rem(my + d, N_DEV)

            @pl.when(lax.div(peer, 2) == b)
            def _(rdma=rs_sends[d]):
                rdma.start()

    reduced = lax.convert_element_type(
        part_ref[pl.ds(my * QROWS, QROWS), :], F32
    )
    for d in (1, 3, 2):
        rs_sends[d].wait_recv()
        reduced = reduced + lax.convert_element_type(rs_ref[d], F32)

    rs_ref[0] = lax.convert_element_type(reduced, BF)
    ag_sends = {}
    for d in (2, 1, 3):
        peer = lax.rem(my + d, N_DEV)
        rdma = pltpu.make_async_remote_copy(
            src_ref=rs_ref.at[0],
            dst_ref=agg_ref.at[pl.ds(my * QROWS, QROWS), :],
            send_sem=ag_send_sems.at[d],
            recv_sem=ag_recv_sems.at[d],
            device_id=(peer,),
            device_id_type=pl.DeviceIdType.MESH,
        )
        rdma.start()
        ag_sends[d] = rdma
    agg_ref[pl.ds(my * QROWS, QROWS), :] = rs_ref[0]

    for d in (1, 3, 2):
        ag_sends[d].wait_recv()
    out_ref[:] = lax.convert_element_type(agg_ref[:], F32)
    out_ref[pl.ds(my * QROWS, QROWS), :] = reduced
    for d in range(1, N_DEV):
        rs_sends[d].wait_send()
        ag_sends[d].wait_send()


def kernel(x, Wq, Wo, K_ext, V_ext):
    my = lax.axis_index("i")
    k_loc = lax.dynamic_slice_in_dim(K_ext, my * H_LOC, H_LOC, axis=2)
    v_loc = lax.dynamic_slice_in_dim(V_ext, my * H_LOC, H_LOC, axis=2)
    kt = jnp.transpose(
        k_loc.astype(BF), (0, 2, 3, 1)
    ).reshape(B * H_LOC, DH, SQ)
    v_t = jnp.transpose(
        v_loc.astype(BF), (0, 2, 1, 3)
    ).reshape(B * H_LOC, SQ, DH)

    out2 = pl.pallas_call(
        _body,
        out_shape=jax.ShapeDtypeStruct((B * SQ, D), F32),
        in_specs=[pl.BlockSpec(memory_space=pltpu.VMEM)] * 5,
        out_specs=pl.BlockSpec(memory_space=pltpu.VMEM),
        scratch_shapes=[
            pltpu.VMEM((B * SQ, D), BF),
            pltpu.VMEM((N_DEV, QROWS, D), BF),
            pltpu.VMEM((B * SQ, D), BF),
            pltpu.SemaphoreType.DMA((N_DEV,)),
            pltpu.SemaphoreType.DMA((N_DEV,)),
            pltpu.SemaphoreType.DMA((N_DEV,)),
            pltpu.SemaphoreType.DMA((N_DEV,)),
        ],
        compiler_params=_CompilerParams(collective_id=0),
    )(x.reshape(B * SQ, D), Wq, Wo, kt, v_t)
    return out2.reshape(B, SQ, D)
